# Initial kernel scaffold; baseline (speedup 1.0000x reference)
import jax
import jax.numpy as jnp
from jax.experimental import pallas as pl


def kernel(prediction, smooth_w, smooth_b):
    raise NotImplementedError("write your pallas kernel here")


# speedup vs baseline: 1.4532x; 1.4532x over previous
import jax
import jax.numpy as jnp
from jax import lax
from jax.experimental import pallas as pl
from jax.experimental.pallas import tpu as pltpu

_GY = 256
_GX = 256
_N_SIGMA = 2
_MIN_OBJ = 10.0


def _shift2d(x, di, dj):
    if di == -1:
        x = jnp.concatenate([x[1:2, :], x[:-1, :]], axis=0)
    elif di == 1:
        x = jnp.concatenate([x[1:, :], x[_GY - 2:_GY - 1, :]], axis=0)
    if dj == -1:
        x = jnp.concatenate([x[:, 1:2], x[:, :-1]], axis=1)
    elif dj == 1:
        x = jnp.concatenate([x[:, 1:], x[:, _GX - 2:_GX - 1]], axis=1)
    return x


def _smooth(x, w_ref, b):
    out = jnp.zeros_like(x)
    for i in range(3):
        for j in range(3):
            out = out + w_ref[i, j] * _shift2d(x, i - 1, j - 1)
    return out + b


def _cluster_body(seed_ref, emb0_ref, emb1_ref, sigr0_ref, sigr1_ref,
                  hist_ref, w_ref, b_ref, inst_ref,
                  sig0_ref, sig1_ref, uncl_ref, dmap_ref):
    b = b_ref[0]
    sig0_ref[:, :] = _smooth(sigr0_ref[:, :], w_ref, b)
    sig1_ref[:, :] = _smooth(sigr1_ref[:, :], w_ref, b)
    sm = _smooth(hist_ref[:, :], w_ref, b) * 9.0
    seed = seed_ref[:, :]
    seeds = (sm >= 4.5) & (seed > 0.1)
    uncl_ref[:, :] = seeds.astype(jnp.float32)
    dmap_ref[:, :] = jnp.zeros((_GY, _GX), jnp.float32)
    inst_ref[:, :] = jnp.zeros((_GY, _GX), jnp.int32)

    mask = seed > 0.5
    iy = lax.broadcasted_iota(jnp.int32, (_GY, _GX), 0)
    ix = lax.broadcasted_iota(jnp.int32, (_GY, _GX), 1)
    lin = iy * _GX + ix
    colsel_i = lax.broadcasted_iota(jnp.int32, (1, _GX), 1)

    def body(carry):
        count, _ = carry
        uncl = uncl_ref[:, :]
        score = seed * uncl
        m = jnp.max(score)
        idx = jnp.min(jnp.where(score == m, lin, jnp.int32(2 ** 30)))
        y = idx // _GX
        x = idx - y * _GX
        csel = (colsel_i == x).astype(jnp.float32)
        c0 = jnp.sum(emb0_ref[pl.ds(y, 1), :] * csel)
        c1 = jnp.sum(emb1_ref[pl.ds(y, 1), :] * csel)
        s0 = jnp.sum(sig0_ref[pl.ds(y, 1), :] * csel)
        s1 = jnp.sum(sig1_ref[pl.ds(y, 1), :] * csel)
        d0 = emb0_ref[:, :] - c0
        d1 = emb1_ref[:, :] - c1
        q = d0 * d0 * s0 + d1 * d1 * s1
        dist = jnp.exp(-1.0 * q)
        dmap = dmap_ref[:, :]
        inst = inst_ref[:, :]
        proposal = (dist > dmap) & (dist > 0.5) & mask
        psum = jnp.sum(proposal.astype(jnp.float32))
        coll = jnp.sum(((inst > 0) & (dist > 0.5)).astype(jnp.float32))
        ok = (psum > 0.0) & (2.0 * coll < psum) & (psum >= _MIN_OBJ)
        okp = ok & proposal
        inst_ref[:, :] = jnp.where(okp, count, inst)
        new_uncl = jnp.where(okp, 0.0, uncl)
        new_uncl = jnp.where(lin == idx, 0.0, new_uncl)
        uncl_ref[:, :] = new_uncl
        dmap_ref[:, :] = jnp.where(proposal, dist, dmap)
        count = count + ok.astype(jnp.int32)
        return (count, jnp.sum(new_uncl))

    nnz0 = jnp.sum(uncl_ref[:, :])
    lax.while_loop(lambda c: c[1] > 0.0, body, (jnp.int32(1), nnz0))


def _cluster_call(seed_map, emb0, emb1, sigr0, sigr1, hist, w, b):
    return pl.pallas_call(
        _cluster_body,
        out_shape=jax.ShapeDtypeStruct((_GY, _GX), jnp.int32),
        in_specs=[
            pl.BlockSpec(memory_space=pltpu.VMEM),
            pl.BlockSpec(memory_space=pltpu.VMEM),
            pl.BlockSpec(memory_space=pltpu.VMEM),
            pl.BlockSpec(memory_space=pltpu.VMEM),
            pl.BlockSpec(memory_space=pltpu.VMEM),
            pl.BlockSpec(memory_space=pltpu.VMEM),
            pl.BlockSpec(memory_space=pltpu.SMEM),
            pl.BlockSpec(memory_space=pltpu.SMEM),
        ],
        scratch_shapes=[
            pltpu.VMEM((_GY, _GX), jnp.float32),
            pltpu.VMEM((_GY, _GX), jnp.float32),
            pltpu.VMEM((_GY, _GX), jnp.float32),
            pltpu.VMEM((_GY, _GX), jnp.float32),
        ],
    )(seed_map, emb0, emb1, sigr0, sigr1, hist, w, b)


def kernel(prediction, smooth_w, smooth_b):
    xm = jnp.broadcast_to(
        jnp.linspace(0.0, 1.0, _GX).reshape(1, 1, -1), (1, _GY, _GX))
    ym = jnp.broadcast_to(
        jnp.linspace(0.0, 1.0, _GY).reshape(1, -1, 1), (1, _GY, _GX))
    yxm = jnp.concatenate([ym, xm], 0).astype(jnp.float32)

    sigr = jnp.exp(jax.nn.sigmoid(prediction[2:2 + _N_SIGMA]) * 10.0)
    seed_map = jax.nn.sigmoid(prediction[2 + _N_SIGMA])
    spatial_emb = jnp.tanh(prediction[0:2]) + yxm

    gs = jnp.array([_GY, _GX], jnp.float32).reshape(2, 1, 1)
    ps = jnp.array([1.0, 1.0], jnp.float32).reshape(2, 1, 1)
    pix = jnp.round(spatial_emb * (gs - 1.0) / ps).astype(jnp.int32)
    mask = seed_map > 0.5
    valid = (mask & jnp.all(pix >= 0, axis=0)
             & (pix[0] <= _GY - 1) & (pix[1] <= _GX - 1))
    py = jnp.clip(pix[0], 0, _GY - 1)
    px = jnp.clip(pix[1], 0, _GX - 1)
    hist = jnp.zeros((_GY, _GX), jnp.float32).at[py, px].add(
        valid.astype(jnp.float32))

    inst = _cluster_call(seed_map, spatial_emb[0], spatial_emb[1],
                         sigr[0], sigr[1], hist, smooth_w, smooth_b)
    return inst.astype(jnp.int16)


# speedup vs baseline: 5.0123x; 3.4493x over previous
import functools

import jax
import jax.numpy as jnp
from jax import lax
from jax.experimental import pallas as pl
from jax.experimental.pallas import tpu as pltpu
from jax.experimental.pallas import tpu_sc as plsc

_GY = 256
_GX = 256
_N_SIGMA = 2
_MIN_OBJ = 10.0

_NPTS = _GY * _GX
_NW = 32
_PTS_PER_W = _NPTS // _NW
_BINS_PER_S = _NPTS // 16

_sc_mesh = plsc.VectorSubcoreMesh(core_axis_name="c", subcore_axis_name="s")


@functools.partial(
    pl.kernel,
    mesh=_sc_mesh,
    out_type=jax.ShapeDtypeStruct((2, _NPTS), jnp.float32),
    scratch_types=[
        pltpu.VMEM((_PTS_PER_W,), jnp.int32),
        pltpu.VMEM((_PTS_PER_W,), jnp.float32),
        pltpu.VMEM((_BINS_PER_S,), jnp.float32),
        pltpu.VMEM_SHARED((_NPTS,), jnp.float32),
    ],
)
def _hist_sc(idx_hbm, val_hbm, out_hbm, idx_v, val_v, zbuf_v, shared):
    c = lax.axis_index("c")
    s = lax.axis_index("s")
    wid = s * 2 + c
    base = wid * _PTS_PER_W

    def _z(i, carry):
        zbuf_v[pl.ds(i * 16, 16)] = jnp.zeros((16,), jnp.float32)
        return carry

    lax.fori_loop(0, _BINS_PER_S // 16, _z, 0)
    pltpu.sync_copy(zbuf_v, shared.at[pl.ds(s * _BINS_PER_S, _BINS_PER_S)])
    plsc.subcore_barrier()
    pltpu.sync_copy(idx_hbm.at[pl.ds(base, _PTS_PER_W)], idx_v)
    pltpu.sync_copy(val_hbm.at[pl.ds(base, _PTS_PER_W)], val_v)
    pltpu.sync_copy(val_v, shared.at[idx_v], add=True)
    plsc.subcore_barrier()
    pltpu.sync_copy(shared.at[pl.ds(s * _BINS_PER_S, _BINS_PER_S)],
                    out_hbm.at[c, pl.ds(s * _BINS_PER_S, _BINS_PER_S)])


def _shift2d(x, di, dj):
    if di == -1:
        x = jnp.concatenate([x[1:2, :], x[:-1, :]], axis=0)
    elif di == 1:
        x = jnp.concatenate([x[1:, :], x[_GY - 2:_GY - 1, :]], axis=0)
    if dj == -1:
        x = jnp.concatenate([x[:, 1:2], x[:, :-1]], axis=1)
    elif dj == 1:
        x = jnp.concatenate([x[:, 1:], x[:, _GX - 2:_GX - 1]], axis=1)
    return x


def _smooth(x, w_ref, b):
    out = jnp.zeros_like(x)
    for i in range(3):
        for j in range(3):
            out = out + w_ref[i, j] * _shift2d(x, i - 1, j - 1)
    return out + b


def _cluster_body(seed_ref, emb0_ref, emb1_ref, sigr0_ref, sigr1_ref,
                  hist_ref, w_ref, b_ref, inst_ref,
                  sig0_ref, sig1_ref, uncl_ref, dmap_ref):
    b = b_ref[0]
    sig0_ref[:, :] = _smooth(sigr0_ref[:, :], w_ref, b)
    sig1_ref[:, :] = _smooth(sigr1_ref[:, :], w_ref, b)
    sm = _smooth(hist_ref[0] + hist_ref[1], w_ref, b) * 9.0
    seed = seed_ref[:, :]
    seeds = (sm >= 4.5) & (seed > 0.1)
    uncl_ref[:, :] = seeds.astype(jnp.float32)
    dmap_ref[:, :] = jnp.zeros((_GY, _GX), jnp.float32)
    inst_ref[:, :] = jnp.zeros((_GY, _GX), jnp.int32)

    mask = seed > 0.5
    iy = lax.broadcasted_iota(jnp.int32, (_GY, _GX), 0)
    ix = lax.broadcasted_iota(jnp.int32, (_GY, _GX), 1)
    lin = iy * _GX + ix
    colsel_i = lax.broadcasted_iota(jnp.int32, (1, _GX), 1)

    def body(carry):
        count, _ = carry
        uncl = uncl_ref[:, :]
        score = seed * uncl
        m = jnp.max(score)
        idx = jnp.min(jnp.where(score == m, lin, jnp.int32(2 ** 30)))
        y = idx // _GX
        x = idx - y * _GX
        csel = (colsel_i == x).astype(jnp.float32)
        c0 = jnp.sum(emb0_ref[pl.ds(y, 1), :] * csel)
        c1 = jnp.sum(emb1_ref[pl.ds(y, 1), :] * csel)
        s0 = jnp.sum(sig0_ref[pl.ds(y, 1), :] * csel)
        s1 = jnp.sum(sig1_ref[pl.ds(y, 1), :] * csel)
        d0 = emb0_ref[:, :] - c0
        d1 = emb1_ref[:, :] - c1
        q = d0 * d0 * s0 + d1 * d1 * s1
        dist = jnp.exp(-1.0 * q)
        dmap = dmap_ref[:, :]
        inst = inst_ref[:, :]
        proposal = (dist > dmap) & (dist > 0.5) & mask
        psum = jnp.sum(proposal.astype(jnp.float32))
        coll = jnp.sum(((inst > 0) & (dist > 0.5)).astype(jnp.float32))
        ok = (psum > 0.0) & (2.0 * coll < psum) & (psum >= _MIN_OBJ)
        okp = ok & proposal
        inst_ref[:, :] = jnp.where(okp, count, inst)
        new_uncl = jnp.where(okp, 0.0, uncl)
        new_uncl = jnp.where(lin == idx, 0.0, new_uncl)
        uncl_ref[:, :] = new_uncl
        dmap_ref[:, :] = jnp.where(proposal, dist, dmap)
        count = count + ok.astype(jnp.int32)
        return (count, jnp.sum(new_uncl))

    nnz0 = jnp.sum(uncl_ref[:, :])
    lax.while_loop(lambda c: c[1] > 0.0, body, (jnp.int32(1), nnz0))


def _cluster_call(seed_map, emb0, emb1, sigr0, sigr1, hist, w, b):
    return pl.pallas_call(
        _cluster_body,
        out_shape=jax.ShapeDtypeStruct((_GY, _GX), jnp.int32),
        in_specs=[
            pl.BlockSpec(memory_space=pltpu.VMEM),
            pl.BlockSpec(memory_space=pltpu.VMEM),
            pl.BlockSpec(memory_space=pltpu.VMEM),
            pl.BlockSpec(memory_space=pltpu.VMEM),
            pl.BlockSpec(memory_space=pltpu.VMEM),
            pl.BlockSpec(memory_space=pltpu.VMEM),
            pl.BlockSpec(memory_space=pltpu.SMEM),
            pl.BlockSpec(memory_space=pltpu.SMEM),
        ],
        scratch_shapes=[
            pltpu.VMEM((_GY, _GX), jnp.float32),
            pltpu.VMEM((_GY, _GX), jnp.float32),
            pltpu.VMEM((_GY, _GX), jnp.float32),
            pltpu.VMEM((_GY, _GX), jnp.float32),
        ],
    )(seed_map, emb0, emb1, sigr0, sigr1, hist, w, b)


def kernel(prediction, smooth_w, smooth_b):
    xm = jnp.broadcast_to(
        jnp.linspace(0.0, 1.0, _GX).reshape(1, 1, -1), (1, _GY, _GX))
    ym = jnp.broadcast_to(
        jnp.linspace(0.0, 1.0, _GY).reshape(1, -1, 1), (1, _GY, _GX))
    yxm = jnp.concatenate([ym, xm], 0).astype(jnp.float32)

    sigr = jnp.exp(jax.nn.sigmoid(prediction[2:2 + _N_SIGMA]) * 10.0)
    seed_map = jax.nn.sigmoid(prediction[2 + _N_SIGMA])
    spatial_emb = jnp.tanh(prediction[0:2]) + yxm

    gs = jnp.array([_GY, _GX], jnp.float32).reshape(2, 1, 1)
    ps = jnp.array([1.0, 1.0], jnp.float32).reshape(2, 1, 1)
    pix = jnp.round(spatial_emb * (gs - 1.0) / ps).astype(jnp.int32)
    mask = seed_map > 0.5
    valid = (mask & jnp.all(pix >= 0, axis=0)
             & (pix[0] <= _GY - 1) & (pix[1] <= _GX - 1))
    py = jnp.clip(pix[0], 0, _GY - 1)
    px = jnp.clip(pix[1], 0, _GX - 1)
    idx = (py * _GX + px).reshape(-1)
    val = valid.astype(jnp.float32).reshape(-1)
    hist = _hist_sc(idx, val).reshape(2, _GY, _GX)

    inst = _cluster_call(seed_map, spatial_emb[0], spatial_emb[1],
                         sigr[0], sigr[1], hist, smooth_w, smooth_b)
    return inst.astype(jnp.int16)


# speedup vs baseline: 5.9354x; 1.1842x over previous
import functools

import jax
import jax.numpy as jnp
from jax import lax
from jax.experimental import pallas as pl
from jax.experimental.pallas import tpu as pltpu
from jax.experimental.pallas import tpu_sc as plsc

_GY = 256
_GX = 256
_N_SIGMA = 2
_MIN_OBJ = 10.0

_NPTS = _GY * _GX
_NDUMMY = 2048
_NBINS = _NPTS + _NDUMMY
_NW = 32
_PTS_PER_W = _NPTS // _NW
_BINS_PER_S = _NBINS // 16
_OUT_PER_S = _NPTS // 16


def _yxm():
    xm = jnp.broadcast_to(
        jnp.linspace(0.0, 1.0, _GX).reshape(1, 1, -1), (1, _GY, _GX))
    ym = jnp.broadcast_to(
        jnp.linspace(0.0, 1.0, _GY).reshape(1, -1, 1), (1, _GY, _GX))
    return jnp.concatenate([ym, xm], 0).astype(jnp.float32)


def _emb_maps(pred_ref, yxm_ref):
    e0 = jnp.tanh(pred_ref[0]) + yxm_ref[0]
    e1 = jnp.tanh(pred_ref[1]) + yxm_ref[1]
    return e0, e1


def _prep_body(pred_ref, yxm_ref, idx_ref):
    e0, e1 = _emb_maps(pred_ref, yxm_ref)
    seed = jax.nn.sigmoid(pred_ref[2 + _N_SIGMA])
    pix0 = jnp.round(e0 * (_GY - 1.0) / 1.0).astype(jnp.int32)
    pix1 = jnp.round(e1 * (_GX - 1.0) / 1.0).astype(jnp.int32)
    valid = ((seed > 0.5) & (pix0 >= 0) & (pix1 >= 0)
             & (pix0 <= _GY - 1) & (pix1 <= _GX - 1))
    py = jnp.clip(pix0, 0, _GY - 1)
    px = jnp.clip(pix1, 0, _GX - 1)
    iy = lax.broadcasted_iota(jnp.int32, (_GY, _GX), 0)
    ix = lax.broadcasted_iota(jnp.int32, (_GY, _GX), 1)
    lin = iy * _GX + ix
    dummy = _NPTS + jnp.bitwise_and(lin, _NDUMMY - 1)
    idx = jnp.where(valid, py * _GX + px, dummy)
    idx_ref[...] = idx.reshape(_NPTS)


def _prep_call(prediction, yxm):
    return pl.pallas_call(
        _prep_body,
        out_shape=jax.ShapeDtypeStruct((_NPTS,), jnp.int32),
        in_specs=[
            pl.BlockSpec(memory_space=pltpu.VMEM),
            pl.BlockSpec(memory_space=pltpu.VMEM),
        ],
    )(prediction, yxm)


_sc_mesh = plsc.VectorSubcoreMesh(core_axis_name="c", subcore_axis_name="s")


@functools.partial(
    pl.kernel,
    mesh=_sc_mesh,
    out_type=jax.ShapeDtypeStruct((2, _NPTS), jnp.float32),
    scratch_types=[
        pltpu.VMEM((_PTS_PER_W,), jnp.int32),
        pltpu.VMEM((_PTS_PER_W,), jnp.float32),
        pltpu.VMEM((_BINS_PER_S,), jnp.float32),
        pltpu.VMEM_SHARED((_NBINS,), jnp.float32),
    ],
)
def _hist_sc(idx_hbm, out_hbm, idx_v, ones_v, zbuf_v, shared):
    c = lax.axis_index("c")
    s = lax.axis_index("s")
    wid = s * 2 + c
    base = wid * _PTS_PER_W

    def _z(i, carry):
        zbuf_v[pl.ds(i * 16, 16)] = jnp.zeros((16,), jnp.float32)
        return carry

    lax.fori_loop(0, _BINS_PER_S // 16, _z, 0)

    def _o(i, carry):
        ones_v[pl.ds(i * 16, 16)] = jnp.ones((16,), jnp.float32)
        return carry

    lax.fori_loop(0, _PTS_PER_W // 16, _o, 0)
    pltpu.sync_copy(zbuf_v, shared.at[pl.ds(s * _BINS_PER_S, _BINS_PER_S)])
    pltpu.sync_copy(idx_hbm.at[pl.ds(base, _PTS_PER_W)], idx_v)
    plsc.subcore_barrier()
    pltpu.sync_copy(ones_v, shared.at[idx_v], add=True)
    plsc.subcore_barrier()
    pltpu.sync_copy(shared.at[pl.ds(s * _OUT_PER_S, _OUT_PER_S)],
                    out_hbm.at[c, pl.ds(s * _OUT_PER_S, _OUT_PER_S)])


def _shift2d(x, di, dj):
    if di == -1:
        x = jnp.concatenate([x[1:2, :], x[:-1, :]], axis=0)
    elif di == 1:
        x = jnp.concatenate([x[1:, :], x[_GY - 2:_GY - 1, :]], axis=0)
    if dj == -1:
        x = jnp.concatenate([x[:, 1:2], x[:, :-1]], axis=1)
    elif dj == 1:
        x = jnp.concatenate([x[:, 1:], x[:, _GX - 2:_GX - 1]], axis=1)
    return x


def _smooth(x, w_ref, b):
    out = jnp.zeros_like(x)
    for i in range(3):
        for j in range(3):
            out = out + w_ref[i, j] * _shift2d(x, i - 1, j - 1)
    return out + b


def _cluster_body(pred_ref, yxm_ref, hist_ref, w_ref, b_ref, out_ref,
                  emb0_ref, emb1_ref, sig0_ref, sig1_ref,
                  inst_ref, uncl_ref, dmap_ref):
    b = b_ref[0]
    e0, e1 = _emb_maps(pred_ref, yxm_ref)
    emb0_ref[:, :] = e0
    emb1_ref[:, :] = e1
    seed = jax.nn.sigmoid(pred_ref[2 + _N_SIGMA])
    sigr0 = jnp.exp(jax.nn.sigmoid(pred_ref[2]) * 10.0)
    sigr1 = jnp.exp(jax.nn.sigmoid(pred_ref[3]) * 10.0)
    sig0_ref[:, :] = _smooth(sigr0, w_ref, b)
    sig1_ref[:, :] = _smooth(sigr1, w_ref, b)
    sm = _smooth(hist_ref[0] + hist_ref[1], w_ref, b) * 9.0
    seeds = (sm >= 4.5) & (seed > 0.1)
    uncl_ref[:, :] = seeds.astype(jnp.float32)
    dmap_ref[:, :] = jnp.zeros((_GY, _GX), jnp.float32)
    inst_ref[:, :] = jnp.zeros((_GY, _GX), jnp.int32)

    mask = seed > 0.5
    iy = lax.broadcasted_iota(jnp.int32, (_GY, _GX), 0)
    ix = lax.broadcasted_iota(jnp.int32, (_GY, _GX), 1)
    lin = iy * _GX + ix
    colsel_i = lax.broadcasted_iota(jnp.int32, (1, _GX), 1)

    def body(carry):
        count, _ = carry
        uncl = uncl_ref[:, :]
        score = seed * uncl
        m = jnp.max(score)
        idx = jnp.min(jnp.where(score == m, lin, jnp.int32(2 ** 30)))
        y = idx // _GX
        x = idx - y * _GX
        csel = (colsel_i == x).astype(jnp.float32)
        c0 = jnp.sum(emb0_ref[pl.ds(y, 1), :] * csel)
        c1 = jnp.sum(emb1_ref[pl.ds(y, 1), :] * csel)
        s0 = jnp.sum(sig0_ref[pl.ds(y, 1), :] * csel)
        s1 = jnp.sum(sig1_ref[pl.ds(y, 1), :] * csel)
        d0 = emb0_ref[:, :] - c0
        d1 = emb1_ref[:, :] - c1
        q = d0 * d0 * s0 + d1 * d1 * s1
        dist = jnp.exp(-1.0 * q)
        dmap = dmap_ref[:, :]
        inst = inst_ref[:, :]
        proposal = (dist > dmap) & (dist > 0.5) & mask
        psum = jnp.sum(proposal.astype(jnp.float32))
        coll = jnp.sum(((inst > 0) & (dist > 0.5)).astype(jnp.float32))
        ok = (psum > 0.0) & (2.0 * coll < psum) & (psum >= _MIN_OBJ)
        okp = ok & proposal
        inst_ref[:, :] = jnp.where(okp, count, inst)
        new_uncl = jnp.where(okp, 0.0, uncl)
        new_uncl = jnp.where(lin == idx, 0.0, new_uncl)
        uncl_ref[:, :] = new_uncl
        dmap_ref[:, :] = jnp.where(proposal, dist, dmap)
        count = count + ok.astype(jnp.int32)
        return (count, jnp.sum(new_uncl))

    nnz0 = jnp.sum(uncl_ref[:, :])
    lax.while_loop(lambda c: c[1] > 0.0, body, (jnp.int32(1), nnz0))
    out_ref[:, :] = inst_ref[:, :].astype(jnp.int16)


def _cluster_call(prediction, yxm, hist, w, b):
    return pl.pallas_call(
        _cluster_body,
        out_shape=jax.ShapeDtypeStruct((_GY, _GX), jnp.int16),
        in_specs=[
            pl.BlockSpec(memory_space=pltpu.VMEM),
            pl.BlockSpec(memory_space=pltpu.VMEM),
            pl.BlockSpec(memory_space=pltpu.VMEM),
            pl.BlockSpec(memory_space=pltpu.SMEM),
            pl.BlockSpec(memory_space=pltpu.SMEM),
        ],
        scratch_shapes=[
            pltpu.VMEM((_GY, _GX), jnp.float32),
            pltpu.VMEM((_GY, _GX), jnp.float32),
            pltpu.VMEM((_GY, _GX), jnp.float32),
            pltpu.VMEM((_GY, _GX), jnp.float32),
            pltpu.VMEM((_GY, _GX), jnp.int32),
            pltpu.VMEM((_GY, _GX), jnp.float32),
            pltpu.VMEM((_GY, _GX), jnp.float32),
        ],
    )(prediction, yxm, hist, w, b)


def kernel(prediction, smooth_w, smooth_b):
    yxm = _yxm()
    idx = _prep_call(prediction, yxm)
    hist = _hist_sc(idx).reshape(2, _GY, _GX)
    return _cluster_call(prediction, yxm, hist, smooth_w, smooth_b)


# speedup vs baseline: 6.3312x; 1.0667x over previous
import functools

import jax
import jax.numpy as jnp
from jax import lax
from jax.experimental import pallas as pl
from jax.experimental.pallas import tpu as pltpu
from jax.experimental.pallas import tpu_sc as plsc

_GY = 256
_GX = 256
_N_SIGMA = 2
_MIN_OBJ = 10.0

_NPTS = _GY * _GX
_NDUMMY = 2048
_NBINS = _NPTS + _NDUMMY
_NW = 32
_PTS_PER_W = _NPTS // _NW
_BINS_PER_S = _NBINS // 16
_OUT_PER_S = _NPTS // 16


def _yxm():
    xm = jnp.broadcast_to(
        jnp.linspace(0.0, 1.0, _GX).reshape(1, 1, -1), (1, _GY, _GX))
    ym = jnp.broadcast_to(
        jnp.linspace(0.0, 1.0, _GY).reshape(1, -1, 1), (1, _GY, _GX))
    return jnp.concatenate([ym, xm], 0).astype(jnp.float32)


def _emb_maps(pred_ref, yxm_ref):
    e0 = jnp.tanh(pred_ref[0]) + yxm_ref[0]
    e1 = jnp.tanh(pred_ref[1]) + yxm_ref[1]
    return e0, e1


def _prep_body(pred_ref, yxm_ref, idx_ref):
    e0, e1 = _emb_maps(pred_ref, yxm_ref)
    seed = jax.nn.sigmoid(pred_ref[2 + _N_SIGMA])
    pix0 = jnp.round(e0 * (_GY - 1.0) / 1.0).astype(jnp.int32)
    pix1 = jnp.round(e1 * (_GX - 1.0) / 1.0).astype(jnp.int32)
    valid = ((seed > 0.5) & (pix0 >= 0) & (pix1 >= 0)
             & (pix0 <= _GY - 1) & (pix1 <= _GX - 1))
    py = jnp.clip(pix0, 0, _GY - 1)
    px = jnp.clip(pix1, 0, _GX - 1)
    iy = lax.broadcasted_iota(jnp.int32, (_GY, _GX), 0)
    ix = lax.broadcasted_iota(jnp.int32, (_GY, _GX), 1)
    lin = iy * _GX + ix
    dummy = _NPTS + jnp.bitwise_and(lin, _NDUMMY - 1)
    idx = jnp.where(valid, py * _GX + px, dummy)
    idx_ref[...] = idx.reshape(_NPTS)


def _prep_call(prediction, yxm):
    return pl.pallas_call(
        _prep_body,
        out_shape=jax.ShapeDtypeStruct((_NPTS,), jnp.int32),
        in_specs=[
            pl.BlockSpec(memory_space=pltpu.VMEM),
            pl.BlockSpec(memory_space=pltpu.VMEM),
        ],
    )(prediction, yxm)


_sc_mesh = plsc.VectorSubcoreMesh(core_axis_name="c", subcore_axis_name="s")


@functools.partial(
    pl.kernel,
    mesh=_sc_mesh,
    out_type=jax.ShapeDtypeStruct((2, _NPTS), jnp.float32),
    scratch_types=[
        pltpu.VMEM((_PTS_PER_W,), jnp.int32),
        pltpu.VMEM((_PTS_PER_W,), jnp.float32),
        pltpu.VMEM((_BINS_PER_S,), jnp.float32),
        pltpu.VMEM_SHARED((_NBINS,), jnp.float32),
    ],
)
def _hist_sc(idx_hbm, out_hbm, idx_v, ones_v, zbuf_v, shared):
    c = lax.axis_index("c")
    s = lax.axis_index("s")
    wid = s * 2 + c
    base = wid * _PTS_PER_W

    def _z(i, carry):
        zbuf_v[pl.ds(i * 16, 16)] = jnp.zeros((16,), jnp.float32)
        return carry

    lax.fori_loop(0, _BINS_PER_S // 16, _z, 0)

    def _o(i, carry):
        ones_v[pl.ds(i * 16, 16)] = jnp.ones((16,), jnp.float32)
        return carry

    lax.fori_loop(0, _PTS_PER_W // 16, _o, 0)
    pltpu.sync_copy(zbuf_v, shared.at[pl.ds(s * _BINS_PER_S, _BINS_PER_S)])
    pltpu.sync_copy(idx_hbm.at[pl.ds(base, _PTS_PER_W)], idx_v)
    plsc.subcore_barrier()
    pltpu.sync_copy(ones_v, shared.at[idx_v], add=True)
    plsc.subcore_barrier()
    pltpu.sync_copy(shared.at[pl.ds(s * _OUT_PER_S, _OUT_PER_S)],
                    out_hbm.at[c, pl.ds(s * _OUT_PER_S, _OUT_PER_S)])


def _shift2d(x, di, dj):
    if di == -1:
        x = jnp.concatenate([x[1:2, :], x[:-1, :]], axis=0)
    elif di == 1:
        x = jnp.concatenate([x[1:, :], x[_GY - 2:_GY - 1, :]], axis=0)
    if dj == -1:
        x = jnp.concatenate([x[:, 1:2], x[:, :-1]], axis=1)
    elif dj == 1:
        x = jnp.concatenate([x[:, 1:], x[:, _GX - 2:_GX - 1]], axis=1)
    return x


def _smooth(x, w_ref, b):
    out = jnp.zeros_like(x)
    for i in range(3):
        for j in range(3):
            out = out + w_ref[i, j] * _shift2d(x, i - 1, j - 1)
    return out + b


def _cluster_body(pred_ref, yxm_ref, hist_ref, w_ref, b_ref, out_ref,
                  emb0_ref, emb1_ref, sig0_ref, sig1_ref,
                  inst_ref, uncl_ref, dmap_ref):
    b = b_ref[0]
    e0, e1 = _emb_maps(pred_ref, yxm_ref)
    emb0_ref[:, :] = e0
    emb1_ref[:, :] = e1
    seed = jax.nn.sigmoid(pred_ref[2 + _N_SIGMA])
    sigr0 = jnp.exp(jax.nn.sigmoid(pred_ref[2]) * 10.0)
    sigr1 = jnp.exp(jax.nn.sigmoid(pred_ref[3]) * 10.0)
    sig0_ref[:, :] = _smooth(sigr0, w_ref, b)
    sig1_ref[:, :] = _smooth(sigr1, w_ref, b)
    hist2d = (hist_ref[0] + hist_ref[1]).reshape(_GY, _GX)
    sm = _smooth(hist2d, w_ref, b) * 9.0
    seeds = (sm >= 4.5) & (seed > 0.1)
    uncl_ref[:, :] = seeds.astype(jnp.float32)
    dmap_ref[:, :] = jnp.zeros((_GY, _GX), jnp.float32)
    inst_ref[:, :] = jnp.zeros((_GY, _GX), jnp.int32)

    mask = seed > 0.5
    iy = lax.broadcasted_iota(jnp.int32, (_GY, _GX), 0)
    ix = lax.broadcasted_iota(jnp.int32, (_GY, _GX), 1)
    lin = iy * _GX + ix
    colsel_i = lax.broadcasted_iota(jnp.int32, (1, _GX), 1)

    def body(carry):
        count, _ = carry
        uncl = uncl_ref[:, :]
        score = seed * uncl
        m = jnp.max(score)
        idx = jnp.min(jnp.where(score == m, lin, jnp.int32(2 ** 30)))
        y = idx // _GX
        x = idx - y * _GX
        csel = (colsel_i == x).astype(jnp.float32)
        c0 = jnp.sum(emb0_ref[pl.ds(y, 1), :] * csel)
        c1 = jnp.sum(emb1_ref[pl.ds(y, 1), :] * csel)
        s0 = jnp.sum(sig0_ref[pl.ds(y, 1), :] * csel)
        s1 = jnp.sum(sig1_ref[pl.ds(y, 1), :] * csel)
        d0 = emb0_ref[:, :] - c0
        d1 = emb1_ref[:, :] - c1
        q = d0 * d0 * s0 + d1 * d1 * s1
        dist = jnp.exp(-1.0 * q)
        dmap = dmap_ref[:, :]
        inst = inst_ref[:, :]
        proposal = (dist > dmap) & (dist > 0.5) & mask
        psum = jnp.sum(proposal.astype(jnp.float32))
        coll = jnp.sum(((inst > 0) & (dist > 0.5)).astype(jnp.float32))
        ok = (psum > 0.0) & (2.0 * coll < psum) & (psum >= _MIN_OBJ)
        okp = ok & proposal
        inst_ref[:, :] = jnp.where(okp, count, inst)
        new_uncl = jnp.where(okp, 0.0, uncl)
        new_uncl = jnp.where(lin == idx, 0.0, new_uncl)
        uncl_ref[:, :] = new_uncl
        dmap_ref[:, :] = jnp.where(proposal, dist, dmap)
        count = count + ok.astype(jnp.int32)
        return (count, jnp.sum(new_uncl))

    nnz0 = jnp.sum(uncl_ref[:, :])
    lax.while_loop(lambda c: c[1] > 0.0, body, (jnp.int32(1), nnz0))
    out_ref[:, :] = inst_ref[:, :].astype(jnp.int16)


def _cluster_call(prediction, yxm, hist, w, b):
    return pl.pallas_call(
        _cluster_body,
        out_shape=jax.ShapeDtypeStruct((_GY, _GX), jnp.int16),
        in_specs=[
            pl.BlockSpec(memory_space=pltpu.VMEM),
            pl.BlockSpec(memory_space=pltpu.VMEM),
            pl.BlockSpec(memory_space=pltpu.VMEM),
            pl.BlockSpec(memory_space=pltpu.SMEM),
            pl.BlockSpec(memory_space=pltpu.SMEM),
        ],
        scratch_shapes=[
            pltpu.VMEM((_GY, _GX), jnp.float32),
            pltpu.VMEM((_GY, _GX), jnp.float32),
            pltpu.VMEM((_GY, _GX), jnp.float32),
            pltpu.VMEM((_GY, _GX), jnp.float32),
            pltpu.VMEM((_GY, _GX), jnp.int32),
            pltpu.VMEM((_GY, _GX), jnp.float32),
            pltpu.VMEM((_GY, _GX), jnp.float32),
        ],
    )(prediction, yxm, hist, w, b)


def kernel(prediction, smooth_w, smooth_b):
    yxm = _yxm()
    idx = _prep_call(prediction, yxm)
    hist = _hist_sc(idx)
    return _cluster_call(prediction, yxm, hist, smooth_w, smooth_b)


# speedup vs baseline: 6.5632x; 1.0366x over previous
import functools

import jax
import jax.numpy as jnp
from jax import lax
from jax.experimental import pallas as pl
from jax.experimental.pallas import tpu as pltpu
from jax.experimental.pallas import tpu_sc as plsc

_GY = 256
_GX = 256
_N_SIGMA = 2
_MIN_OBJ = 10.0

_NPTS = _GY * _GX
_NDUMMY = 2048
_NBINS = _NPTS + _NDUMMY
_NW = 32
_PTS_PER_W = _NPTS // _NW
_BINS_PER_S = _NBINS // 16
_OUT_PER_S = _NPTS // 16


def _emb_maps(pred_ref):
    step = jnp.float32(1.0 / 255.0)
    ym = lax.broadcasted_iota(jnp.int32, (_GY, _GX), 0).astype(jnp.float32)
    xm = lax.broadcasted_iota(jnp.int32, (_GY, _GX), 1).astype(jnp.float32)
    e0 = jnp.tanh(pred_ref[0]) + ym * step
    e1 = jnp.tanh(pred_ref[1]) + xm * step
    return e0, e1


def _prep_body(pred_ref, idx_ref):
    e0, e1 = _emb_maps(pred_ref)
    seed = jax.nn.sigmoid(pred_ref[2 + _N_SIGMA])
    pix0 = jnp.round(e0 * (_GY - 1.0) / 1.0).astype(jnp.int32)
    pix1 = jnp.round(e1 * (_GX - 1.0) / 1.0).astype(jnp.int32)
    valid = ((seed > 0.5) & (pix0 >= 0) & (pix1 >= 0)
             & (pix0 <= _GY - 1) & (pix1 <= _GX - 1))
    py = jnp.clip(pix0, 0, _GY - 1)
    px = jnp.clip(pix1, 0, _GX - 1)
    iy = lax.broadcasted_iota(jnp.int32, (_GY, _GX), 0)
    ix = lax.broadcasted_iota(jnp.int32, (_GY, _GX), 1)
    lin = iy * _GX + ix
    dummy = _NPTS + jnp.bitwise_and(lin, _NDUMMY - 1)
    idx = jnp.where(valid, py * _GX + px, dummy)
    idx_ref[...] = idx.reshape(_NPTS)


def _prep_call(prediction):
    return pl.pallas_call(
        _prep_body,
        out_shape=jax.ShapeDtypeStruct((_NPTS,), jnp.int32),
        in_specs=[
            pl.BlockSpec(memory_space=pltpu.VMEM),
        ],
    )(prediction)


_sc_mesh = plsc.VectorSubcoreMesh(core_axis_name="c", subcore_axis_name="s")


@functools.partial(
    pl.kernel,
    mesh=_sc_mesh,
    out_type=jax.ShapeDtypeStruct((2, _NPTS), jnp.float32),
    scratch_types=[
        pltpu.VMEM((_PTS_PER_W,), jnp.int32),
        pltpu.VMEM((_PTS_PER_W,), jnp.float32),
        pltpu.VMEM((_BINS_PER_S,), jnp.float32),
        pltpu.VMEM_SHARED((_NBINS,), jnp.float32),
    ],
)
def _hist_sc(idx_hbm, out_hbm, idx_v, ones_v, zbuf_v, shared):
    c = lax.axis_index("c")
    s = lax.axis_index("s")
    wid = s * 2 + c
    base = wid * _PTS_PER_W

    def _z(i, carry):
        zbuf_v[pl.ds(i * 16, 16)] = jnp.zeros((16,), jnp.float32)
        return carry

    lax.fori_loop(0, _BINS_PER_S // 16, _z, 0)

    def _o(i, carry):
        ones_v[pl.ds(i * 16, 16)] = jnp.ones((16,), jnp.float32)
        return carry

    lax.fori_loop(0, _PTS_PER_W // 16, _o, 0)
    pltpu.sync_copy(zbuf_v, shared.at[pl.ds(s * _BINS_PER_S, _BINS_PER_S)])
    pltpu.sync_copy(idx_hbm.at[pl.ds(base, _PTS_PER_W)], idx_v)
    plsc.subcore_barrier()
    pltpu.sync_copy(ones_v, shared.at[idx_v], add=True)
    plsc.subcore_barrier()
    pltpu.sync_copy(shared.at[pl.ds(s * _OUT_PER_S, _OUT_PER_S)],
                    out_hbm.at[c, pl.ds(s * _OUT_PER_S, _OUT_PER_S)])


def _shift2d(x, di, dj):
    if di == -1:
        x = jnp.concatenate([x[1:2, :], x[:-1, :]], axis=0)
    elif di == 1:
        x = jnp.concatenate([x[1:, :], x[_GY - 2:_GY - 1, :]], axis=0)
    if dj == -1:
        x = jnp.concatenate([x[:, 1:2], x[:, :-1]], axis=1)
    elif dj == 1:
        x = jnp.concatenate([x[:, 1:], x[:, _GX - 2:_GX - 1]], axis=1)
    return x


def _smooth(x, w_ref, b):
    out = jnp.zeros_like(x)
    for i in range(3):
        for j in range(3):
            out = out + w_ref[i, j] * _shift2d(x, i - 1, j - 1)
    return out + b


def _cluster_body(pred_ref, hist_ref, w_ref, b_ref, out_ref,
                  emb0_ref, emb1_ref, sig0_ref, sig1_ref,
                  inst_ref, uncl_ref, dmap_ref):
    b = b_ref[0]
    e0, e1 = _emb_maps(pred_ref)
    emb0_ref[:, :] = e0
    emb1_ref[:, :] = e1
    seed = jax.nn.sigmoid(pred_ref[2 + _N_SIGMA])
    sigr0 = jnp.exp(jax.nn.sigmoid(pred_ref[2]) * 10.0)
    sigr1 = jnp.exp(jax.nn.sigmoid(pred_ref[3]) * 10.0)
    sig0_ref[:, :] = _smooth(sigr0, w_ref, b)
    sig1_ref[:, :] = _smooth(sigr1, w_ref, b)
    hist2d = (hist_ref[0] + hist_ref[1]).reshape(_GY, _GX)
    sm = _smooth(hist2d, w_ref, b) * 9.0
    seeds = (sm >= 4.5) & (seed > 0.1)
    uncl_ref[:, :] = seeds.astype(jnp.float32)
    dmap_ref[:, :] = jnp.zeros((_GY, _GX), jnp.float32)
    inst_ref[:, :] = jnp.zeros((_GY, _GX), jnp.int32)

    mask = seed > 0.5
    iy = lax.broadcasted_iota(jnp.int32, (_GY, _GX), 0)
    ix = lax.broadcasted_iota(jnp.int32, (_GY, _GX), 1)
    lin = iy * _GX + ix
    colsel_i = lax.broadcasted_iota(jnp.int32, (1, _GX), 1)

    def body(carry):
        count, _ = carry
        uncl = uncl_ref[:, :]
        score = seed * uncl
        m = jnp.max(score)
        idx = jnp.min(jnp.where(score == m, lin, jnp.int32(2 ** 30)))
        y = idx // _GX
        x = idx - y * _GX
        csel = (colsel_i == x).astype(jnp.float32)
        c0 = jnp.sum(emb0_ref[pl.ds(y, 1), :] * csel)
        c1 = jnp.sum(emb1_ref[pl.ds(y, 1), :] * csel)
        s0 = jnp.sum(sig0_ref[pl.ds(y, 1), :] * csel)
        s1 = jnp.sum(sig1_ref[pl.ds(y, 1), :] * csel)
        d0 = emb0_ref[:, :] - c0
        d1 = emb1_ref[:, :] - c1
        q = d0 * d0 * s0 + d1 * d1 * s1
        dist = jnp.exp(-1.0 * q)
        dmap = dmap_ref[:, :]
        inst = inst_ref[:, :]
        proposal = (dist > dmap) & (dist > 0.5) & mask
        psum = jnp.sum(proposal.astype(jnp.float32))
        coll = jnp.sum(((inst > 0) & (dist > 0.5)).astype(jnp.float32))
        ok = (psum > 0.0) & (2.0 * coll < psum) & (psum >= _MIN_OBJ)
        okp = ok & proposal
        inst_ref[:, :] = jnp.where(okp, count, inst)
        new_uncl = jnp.where(okp, 0.0, uncl)
        new_uncl = jnp.where(lin == idx, 0.0, new_uncl)
        uncl_ref[:, :] = new_uncl
        dmap_ref[:, :] = jnp.where(proposal, dist, dmap)
        count = count + ok.astype(jnp.int32)
        return (count, jnp.sum(new_uncl))

    nnz0 = jnp.sum(uncl_ref[:, :])
    lax.while_loop(lambda c: c[1] > 0.0, body, (jnp.int32(1), nnz0))
    out_ref[:, :] = inst_ref[:, :].astype(jnp.int16)


def _cluster_call(prediction, hist, w, b):
    return pl.pallas_call(
        _cluster_body,
        out_shape=jax.ShapeDtypeStruct((_GY, _GX), jnp.int16),
        in_specs=[
            pl.BlockSpec(memory_space=pltpu.VMEM),
            pl.BlockSpec(memory_space=pltpu.VMEM),
            pl.BlockSpec(memory_space=pltpu.SMEM),
            pl.BlockSpec(memory_space=pltpu.SMEM),
        ],
        scratch_shapes=[
            pltpu.VMEM((_GY, _GX), jnp.float32),
            pltpu.VMEM((_GY, _GX), jnp.float32),
            pltpu.VMEM((_GY, _GX), jnp.float32),
            pltpu.VMEM((_GY, _GX), jnp.float32),
            pltpu.VMEM((_GY, _GX), jnp.int32),
            pltpu.VMEM((_GY, _GX), jnp.float32),
            pltpu.VMEM((_GY, _GX), jnp.float32),
        ],
    )(prediction, hist, w, b)


def kernel(prediction, smooth_w, smooth_b):
    idx = _prep_call(prediction)
    hist = _hist_sc(idx)
    return _cluster_call(prediction, hist, smooth_w, smooth_b)


# speedup vs baseline: 6.6734x; 1.0168x over previous
import functools

import jax
import jax.numpy as jnp
from jax import lax
from jax.experimental import pallas as pl
from jax.experimental.pallas import tpu as pltpu
from jax.experimental.pallas import tpu_sc as plsc

_GY = 256
_GX = 256
_N_SIGMA = 2
_MIN_OBJ = 10.0

_NPTS = _GY * _GX
_NDUMMY = 2048
_NBINS = _NPTS + _NDUMMY
_NW = 16
_PTS_PER_W = _NPTS // _NW
_BINS_PER_S = _NBINS // 16
_OUT_PER_S = _NPTS // 16


def _emb_maps(pred_ref):
    step = jnp.float32(1.0 / 255.0)
    ym = lax.broadcasted_iota(jnp.int32, (_GY, _GX), 0).astype(jnp.float32)
    xm = lax.broadcasted_iota(jnp.int32, (_GY, _GX), 1).astype(jnp.float32)
    e0 = jnp.tanh(pred_ref[0]) + ym * step
    e1 = jnp.tanh(pred_ref[1]) + xm * step
    return e0, e1


def _prep_body(pred_ref, idx_ref):
    e0, e1 = _emb_maps(pred_ref)
    seed = jax.nn.sigmoid(pred_ref[2 + _N_SIGMA])
    pix0 = jnp.round(e0 * (_GY - 1.0) / 1.0).astype(jnp.int32)
    pix1 = jnp.round(e1 * (_GX - 1.0) / 1.0).astype(jnp.int32)
    valid = ((seed > 0.5) & (pix0 >= 0) & (pix1 >= 0)
             & (pix0 <= _GY - 1) & (pix1 <= _GX - 1))
    py = jnp.clip(pix0, 0, _GY - 1)
    px = jnp.clip(pix1, 0, _GX - 1)
    iy = lax.broadcasted_iota(jnp.int32, (_GY, _GX), 0)
    ix = lax.broadcasted_iota(jnp.int32, (_GY, _GX), 1)
    lin = iy * _GX + ix
    dummy = _NPTS + jnp.bitwise_and(lin, _NDUMMY - 1)
    idx = jnp.where(valid, py * _GX + px, dummy)
    idx_ref[...] = idx.reshape(_NPTS)


def _prep_call(prediction):
    return pl.pallas_call(
        _prep_body,
        out_shape=jax.ShapeDtypeStruct((_NPTS,), jnp.int32),
        in_specs=[
            pl.BlockSpec(memory_space=pltpu.VMEM),
        ],
    )(prediction)


_sc_mesh = plsc.VectorSubcoreMesh(core_axis_name="c", subcore_axis_name="s",
                                  num_cores=1)


@functools.partial(
    pl.kernel,
    mesh=_sc_mesh,
    out_type=jax.ShapeDtypeStruct((1, _NPTS), jnp.float32),
    scratch_types=[
        pltpu.VMEM((_PTS_PER_W,), jnp.int32),
        pltpu.VMEM((_PTS_PER_W,), jnp.float32),
        pltpu.VMEM((_BINS_PER_S,), jnp.float32),
        pltpu.VMEM_SHARED((_NBINS,), jnp.float32),
    ],
)
def _hist_sc(idx_hbm, out_hbm, idx_v, ones_v, zbuf_v, shared):
    c = lax.axis_index("c")
    s = lax.axis_index("s")
    wid = s
    base = wid * _PTS_PER_W

    def _z(i, carry):
        zbuf_v[pl.ds(i * 16, 16)] = jnp.zeros((16,), jnp.float32)
        return carry

    lax.fori_loop(0, _BINS_PER_S // 16, _z, 0)

    def _o(i, carry):
        ones_v[pl.ds(i * 16, 16)] = jnp.ones((16,), jnp.float32)
        return carry

    lax.fori_loop(0, _PTS_PER_W // 16, _o, 0)
    pltpu.sync_copy(zbuf_v, shared.at[pl.ds(s * _BINS_PER_S, _BINS_PER_S)])
    pltpu.sync_copy(idx_hbm.at[pl.ds(base, _PTS_PER_W)], idx_v)
    plsc.subcore_barrier()
    pltpu.sync_copy(ones_v, shared.at[idx_v], add=True)
    plsc.subcore_barrier()
    pltpu.sync_copy(shared.at[pl.ds(s * _OUT_PER_S, _OUT_PER_S)],
                    out_hbm.at[c, pl.ds(s * _OUT_PER_S, _OUT_PER_S)])


def _shift2d(x, di, dj):
    if di == -1:
        x = jnp.concatenate([x[1:2, :], x[:-1, :]], axis=0)
    elif di == 1:
        x = jnp.concatenate([x[1:, :], x[_GY - 2:_GY - 1, :]], axis=0)
    if dj == -1:
        x = jnp.concatenate([x[:, 1:2], x[:, :-1]], axis=1)
    elif dj == 1:
        x = jnp.concatenate([x[:, 1:], x[:, _GX - 2:_GX - 1]], axis=1)
    return x


def _smooth(x, w_ref, b):
    out = jnp.zeros_like(x)
    for i in range(3):
        for j in range(3):
            out = out + w_ref[i, j] * _shift2d(x, i - 1, j - 1)
    return out + b


def _cluster_body(pred_ref, hist_ref, w_ref, b_ref, out_ref,
                  emb0_ref, emb1_ref, sig0_ref, sig1_ref,
                  inst_ref, uncl_ref, dmap_ref):
    b = b_ref[0]
    e0, e1 = _emb_maps(pred_ref)
    emb0_ref[:, :] = e0
    emb1_ref[:, :] = e1
    seed = jax.nn.sigmoid(pred_ref[2 + _N_SIGMA])
    sigr0 = jnp.exp(jax.nn.sigmoid(pred_ref[2]) * 10.0)
    sigr1 = jnp.exp(jax.nn.sigmoid(pred_ref[3]) * 10.0)
    sig0_ref[:, :] = _smooth(sigr0, w_ref, b)
    sig1_ref[:, :] = _smooth(sigr1, w_ref, b)
    hist2d = hist_ref[0].reshape(_GY, _GX)
    sm = _smooth(hist2d, w_ref, b) * 9.0
    seeds = (sm >= 4.5) & (seed > 0.1)
    uncl_ref[:, :] = seeds.astype(jnp.float32)
    dmap_ref[:, :] = jnp.zeros((_GY, _GX), jnp.float32)
    inst_ref[:, :] = jnp.zeros((_GY, _GX), jnp.int32)

    mask = seed > 0.5
    iy = lax.broadcasted_iota(jnp.int32, (_GY, _GX), 0)
    ix = lax.broadcasted_iota(jnp.int32, (_GY, _GX), 1)
    lin = iy * _GX + ix
    colsel_i = lax.broadcasted_iota(jnp.int32, (1, _GX), 1)

    def body(carry):
        count, _ = carry
        uncl = uncl_ref[:, :]
        score = seed * uncl
        m = jnp.max(score)
        idx = jnp.min(jnp.where(score == m, lin, jnp.int32(2 ** 30)))
        y = idx // _GX
        x = idx - y * _GX
        csel = (colsel_i == x).astype(jnp.float32)
        c0 = jnp.sum(emb0_ref[pl.ds(y, 1), :] * csel)
        c1 = jnp.sum(emb1_ref[pl.ds(y, 1), :] * csel)
        s0 = jnp.sum(sig0_ref[pl.ds(y, 1), :] * csel)
        s1 = jnp.sum(sig1_ref[pl.ds(y, 1), :] * csel)
        d0 = emb0_ref[:, :] - c0
        d1 = emb1_ref[:, :] - c1
        q = d0 * d0 * s0 + d1 * d1 * s1
        dist = jnp.exp(-1.0 * q)
        dmap = dmap_ref[:, :]
        inst = inst_ref[:, :]
        proposal = (dist > dmap) & (dist > 0.5) & mask
        psum = jnp.sum(proposal.astype(jnp.float32))
        coll = jnp.sum(((inst > 0) & (dist > 0.5)).astype(jnp.float32))
        ok = (psum > 0.0) & (2.0 * coll < psum) & (psum >= _MIN_OBJ)
        okp = ok & proposal
        inst_ref[:, :] = jnp.where(okp, count, inst)
        new_uncl = jnp.where(okp, 0.0, uncl)
        new_uncl = jnp.where(lin == idx, 0.0, new_uncl)
        uncl_ref[:, :] = new_uncl
        dmap_ref[:, :] = jnp.where(proposal, dist, dmap)
        count = count + ok.astype(jnp.int32)
        return (count, jnp.sum(new_uncl))

    nnz0 = jnp.sum(uncl_ref[:, :])
    lax.while_loop(lambda c: c[1] > 0.0, body, (jnp.int32(1), nnz0))
    out_ref[:, :] = inst_ref[:, :].astype(jnp.int16)


def _cluster_call(prediction, hist, w, b):
    return pl.pallas_call(
        _cluster_body,
        out_shape=jax.ShapeDtypeStruct((_GY, _GX), jnp.int16),
        in_specs=[
            pl.BlockSpec(memory_space=pltpu.VMEM),
            pl.BlockSpec(memory_space=pltpu.VMEM),
            pl.BlockSpec(memory_space=pltpu.SMEM),
            pl.BlockSpec(memory_space=pltpu.SMEM),
        ],
        scratch_shapes=[
            pltpu.VMEM((_GY, _GX), jnp.float32),
            pltpu.VMEM((_GY, _GX), jnp.float32),
            pltpu.VMEM((_GY, _GX), jnp.float32),
            pltpu.VMEM((_GY, _GX), jnp.float32),
            pltpu.VMEM((_GY, _GX), jnp.int32),
            pltpu.VMEM((_GY, _GX), jnp.float32),
            pltpu.VMEM((_GY, _GX), jnp.float32),
        ],
    )(prediction, hist, w, b)


def kernel(prediction, smooth_w, smooth_b):
    idx = _prep_call(prediction)
    hist = _hist_sc(idx)
    return _cluster_call(prediction, hist, smooth_w, smooth_b)


# speedup vs baseline: 7.3083x; 1.0951x over previous
import functools

import jax
import jax.numpy as jnp
from jax import lax
from jax.experimental import pallas as pl
from jax.experimental.pallas import tpu as pltpu
from jax.experimental.pallas import tpu_sc as plsc

_GY = 256
_GX = 256
_N_SIGMA = 2
_MIN_OBJ = 10.0

_NPTS = _GY * _GX
_NDUMMY = 2048
_NBINS = _NPTS + _NDUMMY
_NW = 16
_PTS_PER_W = _NPTS // _NW
_BINS_PER_S = _NBINS // 16
_OUT_PER_S = _NPTS // 16


def _emb_maps(pred_ref):
    step = jnp.float32(1.0 / 255.0)
    ym = lax.broadcasted_iota(jnp.int32, (_GY, _GX), 0).astype(jnp.float32)
    xm = lax.broadcasted_iota(jnp.int32, (_GY, _GX), 1).astype(jnp.float32)
    e0 = jnp.tanh(pred_ref[0]) + ym * step
    e1 = jnp.tanh(pred_ref[1]) + xm * step
    return e0, e1


def _prep_body(pred_ref, idx_ref):
    e0, e1 = _emb_maps(pred_ref)
    seed = jax.nn.sigmoid(pred_ref[2 + _N_SIGMA])
    pix0 = jnp.round(e0 * (_GY - 1.0) / 1.0).astype(jnp.int32)
    pix1 = jnp.round(e1 * (_GX - 1.0) / 1.0).astype(jnp.int32)
    valid = ((seed > 0.5) & (pix0 >= 0) & (pix1 >= 0)
             & (pix0 <= _GY - 1) & (pix1 <= _GX - 1))
    py = jnp.clip(pix0, 0, _GY - 1)
    px = jnp.clip(pix1, 0, _GX - 1)
    iy = lax.broadcasted_iota(jnp.int32, (_GY, _GX), 0)
    ix = lax.broadcasted_iota(jnp.int32, (_GY, _GX), 1)
    lin = iy * _GX + ix
    dummy = _NPTS + jnp.bitwise_and(lin, _NDUMMY - 1)
    idx = jnp.where(valid, py * _GX + px, dummy)
    idx_ref[...] = idx.reshape(_NPTS)


def _prep_call(prediction):
    return pl.pallas_call(
        _prep_body,
        out_shape=jax.ShapeDtypeStruct((_NPTS,), jnp.int32),
        in_specs=[
            pl.BlockSpec(memory_space=pltpu.VMEM),
        ],
    )(prediction)


_sc_mesh = plsc.VectorSubcoreMesh(core_axis_name="c", subcore_axis_name="s",
                                  num_cores=1)


@functools.partial(
    pl.kernel,
    mesh=_sc_mesh,
    out_type=jax.ShapeDtypeStruct((1, _NPTS), jnp.float32),
    scratch_types=[
        pltpu.VMEM((_PTS_PER_W,), jnp.int32),
        pltpu.VMEM((_PTS_PER_W,), jnp.float32),
        pltpu.VMEM((_BINS_PER_S,), jnp.float32),
        pltpu.VMEM_SHARED((_NBINS,), jnp.float32),
        pltpu.SemaphoreType.DMA,
    ],
)
def _hist_sc(idx_hbm, out_hbm, idx_v, ones_v, zbuf_v, shared, sem):
    c = lax.axis_index("c")
    s = lax.axis_index("s")
    base = s * _PTS_PER_W

    idx_cp = pltpu.async_copy(idx_hbm.at[pl.ds(base, _PTS_PER_W)], idx_v, sem)

    def _z(i, carry):
        for j in range(8):
            zbuf_v[pl.ds(i * 128 + j * 16, 16)] = jnp.zeros((16,), jnp.float32)
        return carry

    lax.fori_loop(0, _BINS_PER_S // 128, _z, 0)

    def _o(i, carry):
        for j in range(8):
            ones_v[pl.ds(i * 128 + j * 16, 16)] = jnp.ones((16,), jnp.float32)
        return carry

    lax.fori_loop(0, _PTS_PER_W // 128, _o, 0)
    pltpu.sync_copy(zbuf_v, shared.at[pl.ds(s * _BINS_PER_S, _BINS_PER_S)])
    idx_cp.wait()
    plsc.subcore_barrier()
    pltpu.sync_copy(ones_v, shared.at[idx_v], add=True)
    plsc.subcore_barrier()
    pltpu.sync_copy(shared.at[pl.ds(s * _OUT_PER_S, _OUT_PER_S)],
                    out_hbm.at[c, pl.ds(s * _OUT_PER_S, _OUT_PER_S)])


def _shift2d(x, di, dj):
    if di == -1:
        x = jnp.concatenate([x[1:2, :], x[:-1, :]], axis=0)
    elif di == 1:
        x = jnp.concatenate([x[1:, :], x[_GY - 2:_GY - 1, :]], axis=0)
    if dj == -1:
        x = jnp.concatenate([x[:, 1:2], x[:, :-1]], axis=1)
    elif dj == 1:
        x = jnp.concatenate([x[:, 1:], x[:, _GX - 2:_GX - 1]], axis=1)
    return x


def _smooth(x, w_ref, b):
    out = jnp.zeros_like(x)
    for i in range(3):
        for j in range(3):
            out = out + w_ref[i, j] * _shift2d(x, i - 1, j - 1)
    return out + b


def _cluster_body(pred_ref, hist_ref, w_ref, b_ref, out_ref,
                  emb0_ref, emb1_ref, sig0_ref, sig1_ref,
                  inst_ref, uncl_ref, dmap_ref):
    b = b_ref[0]
    e0, e1 = _emb_maps(pred_ref)
    emb0_ref[:, :] = e0
    emb1_ref[:, :] = e1
    seed = jax.nn.sigmoid(pred_ref[2 + _N_SIGMA])
    sigr0 = jnp.exp(jax.nn.sigmoid(pred_ref[2]) * 10.0)
    sigr1 = jnp.exp(jax.nn.sigmoid(pred_ref[3]) * 10.0)
    sig0_ref[:, :] = _smooth(sigr0, w_ref, b)
    sig1_ref[:, :] = _smooth(sigr1, w_ref, b)
    hist2d = hist_ref[0].reshape(_GY, _GX)
    sm = _smooth(hist2d, w_ref, b) * 9.0
    seeds = (sm >= 4.5) & (seed > 0.1)
    uncl_ref[:, :] = seeds.astype(jnp.float32)
    dmap_ref[:, :] = jnp.zeros((_GY, _GX), jnp.float32)
    inst_ref[:, :] = jnp.zeros((_GY, _GX), jnp.int32)

    mask = seed > 0.5
    iy = lax.broadcasted_iota(jnp.int32, (_GY, _GX), 0)
    ix = lax.broadcasted_iota(jnp.int32, (_GY, _GX), 1)
    lin = iy * _GX + ix
    colsel_i = lax.broadcasted_iota(jnp.int32, (1, _GX), 1)

    def body(carry):
        count, _ = carry
        uncl = uncl_ref[:, :]
        score = seed * uncl
        m = jnp.max(score)
        idx = jnp.min(jnp.where(score == m, lin, jnp.int32(2 ** 30)))
        y = idx // _GX
        x = idx - y * _GX
        csel = (colsel_i == x).astype(jnp.float32)
        c0 = jnp.sum(emb0_ref[pl.ds(y, 1), :] * csel)
        c1 = jnp.sum(emb1_ref[pl.ds(y, 1), :] * csel)
        s0 = jnp.sum(sig0_ref[pl.ds(y, 1), :] * csel)
        s1 = jnp.sum(sig1_ref[pl.ds(y, 1), :] * csel)
        d0 = emb0_ref[:, :] - c0
        d1 = emb1_ref[:, :] - c1
        q = d0 * d0 * s0 + d1 * d1 * s1
        dist = jnp.exp(-1.0 * q)
        dmap = dmap_ref[:, :]
        inst = inst_ref[:, :]
        proposal = (dist > dmap) & (dist > 0.5) & mask
        psum = jnp.sum(proposal.astype(jnp.float32))
        coll = jnp.sum(((inst > 0) & (dist > 0.5)).astype(jnp.float32))
        ok = (psum > 0.0) & (2.0 * coll < psum) & (psum >= _MIN_OBJ)
        okp = ok & proposal
        inst_ref[:, :] = jnp.where(okp, count, inst)
        new_uncl = jnp.where(okp, 0.0, uncl)
        new_uncl = jnp.where(lin == idx, 0.0, new_uncl)
        uncl_ref[:, :] = new_uncl
        dmap_ref[:, :] = jnp.where(proposal, dist, dmap)
        count = count + ok.astype(jnp.int32)
        return (count, jnp.sum(new_uncl))

    nnz0 = jnp.sum(uncl_ref[:, :])
    lax.while_loop(lambda c: c[1] > 0.0, body, (jnp.int32(1), nnz0))
    out_ref[:, :] = inst_ref[:, :].astype(jnp.int16)


def _cluster_call(prediction, hist, w, b):
    return pl.pallas_call(
        _cluster_body,
        out_shape=jax.ShapeDtypeStruct((_GY, _GX), jnp.int16),
        in_specs=[
            pl.BlockSpec(memory_space=pltpu.VMEM),
            pl.BlockSpec(memory_space=pltpu.VMEM),
            pl.BlockSpec(memory_space=pltpu.SMEM),
            pl.BlockSpec(memory_space=pltpu.SMEM),
        ],
        scratch_shapes=[
            pltpu.VMEM((_GY, _GX), jnp.float32),
            pltpu.VMEM((_GY, _GX), jnp.float32),
            pltpu.VMEM((_GY, _GX), jnp.float32),
            pltpu.VMEM((_GY, _GX), jnp.float32),
            pltpu.VMEM((_GY, _GX), jnp.int32),
            pltpu.VMEM((_GY, _GX), jnp.float32),
            pltpu.VMEM((_GY, _GX), jnp.float32),
        ],
    )(prediction, hist, w, b)


def kernel(prediction, smooth_w, smooth_b):
    idx = _prep_call(prediction)
    hist = _hist_sc(idx)
    return _cluster_call(prediction, hist, smooth_w, smooth_b)
